# baseline (speedup 1.0000x reference)
import functools
import jax
import jax.numpy as jnp
from jax import lax
from jax.experimental import pallas as pl
from jax.experimental.pallas import tpu as pltpu
from jax.experimental.pallas import tpu_sc as plsc

D = 64
NC, NS, L = 2, 16, 16
NW = NC * NS
SCALE = 8.0
PPS = 2
NBUF = 2


def _make_kernel(BATCH, SEQ):
    B = BATCH * SEQ
    b_per_w = B // NW
    p_per_w = BATCH // NW
    CHUNK = PPS * SEQ
    n_steps = p_per_w // PPS
    assert n_steps * PPS == p_per_w and n_steps % NBUF == 0
    n_rounds = n_steps // NBUF
    mesh = plsc.VectorSubcoreMesh(
        core_axis_name="c", subcore_axis_name="s",
        num_cores=NC, num_subcores=NS,
    )

    scratch = dict(
        idx_all=pltpu.VMEM((b_per_w,), jnp.int32),
        gbuf=[pltpu.VMEM((CHUNK, D), jnp.float32) for _ in range(NBUF)],
        sbuf=[pltpu.VMEM((PPS, SEQ, D), jnp.float32) for _ in range(NBUF)],
        gsem=[pltpu.SemaphoreType.DMA for _ in range(NBUF)],
        ssem=[pltpu.SemaphoreType.DMA for _ in range(NBUF)],
    )

    @functools.partial(
        pl.kernel,
        mesh=mesh,
        compiler_params=pltpu.CompilerParams(use_tc_tiling_on_sc=False),
        out_type=jax.ShapeDtypeStruct((BATCH, SEQ, D), jnp.float32),
        scratch_types=scratch,
    )
    def k(x_hbm, table_hbm, out_hbm, idx_all, gbuf, sbuf, gsem, ssem):
        wid = lax.axis_index("s") * NC + lax.axis_index("c")
        rbase = wid * b_per_w
        pbase = wid * p_per_w

        pltpu.sync_copy(x_hbm.at[pl.ds(rbase, b_per_w)], idx_all)

        def issue_gather(c, b):
            pltpu.async_copy(
                table_hbm.at[idx_all.at[pl.ds(c * CHUNK, CHUNK)]],
                gbuf[b], gsem[b])

        def issue_store(c, b):
            pltpu.async_copy(
                sbuf[b], out_hbm.at[pl.ds(pbase + c * PPS, PPS)], ssem[b])

        def wait_gather(b):
            pltpu.make_async_copy(table_hbm.at[idx_all.at[pl.ds(0, CHUNK)]],
                                  gbuf[b], gsem[b]).wait()

        def wait_store(b):
            pltpu.make_async_copy(sbuf[b], out_hbm.at[pl.ds(0, PPS)],
                                  ssem[b]).wait()

        def scale(b):
            for p in range(PPS):
                def row_body(s, carry):
                    for j in range(D // L):
                        sl = pl.ds(j * L, L)
                        sbuf[b][p, s, sl] = gbuf[b][p * SEQ + s, sl] * SCALE
                    return carry
                lax.fori_loop(0, SEQ, row_body, 0, unroll=8)

        for b in range(NBUF):
            issue_gather(b, b)

        for b in range(NBUF):
            wait_gather(b)
            scale(b)
            issue_gather(NBUF + b, b)
            issue_store(b, b)

        def round_body(r, carry):
            c0 = r * NBUF
            for b in range(NBUF):
                c = c0 + b
                wait_gather(b)
                wait_store(b)
                scale(b)
                issue_gather(c + NBUF, b)
                issue_store(c, b)
            return carry
        lax.fori_loop(1, n_rounds - 1, round_body, 0)

        for b in range(NBUF):
            c = (n_rounds - 1) * NBUF + b
            wait_gather(b)
            wait_store(b)
            scale(b)
            issue_store(c, b)
        for b in range(NBUF):
            wait_store(b)

    return k


def kernel(x, table):
    BATCH, SEQ = x.shape
    xf = x.reshape(BATCH * SEQ).astype(jnp.int32)
    return _make_kernel(BATCH, SEQ)(xf, table)


# speedup vs baseline: 1.2669x; 1.2669x over previous
import functools
import jax
import jax.numpy as jnp
from jax import lax
from jax.experimental import pallas as pl
from jax.experimental.pallas import tpu as pltpu
from jax.experimental.pallas import tpu_sc as plsc

D = 64
NC, NS, L = 2, 16, 16
NW = NC * NS
SCALE = 8.0
PPS = 2
NBUF = 2


def _make_kernel(BATCH, SEQ):
    B = BATCH * SEQ
    b_per_w = B // NW
    p_per_w = BATCH // NW
    CHUNK = PPS * SEQ
    n_steps = p_per_w // PPS
    assert n_steps * PPS == p_per_w and n_steps % NBUF == 0
    n_rounds = n_steps // NBUF
    mesh = plsc.VectorSubcoreMesh(
        core_axis_name="c", subcore_axis_name="s",
        num_cores=NC, num_subcores=NS,
    )

    scratch = dict(
        idx_all=pltpu.VMEM((b_per_w,), jnp.int32),
        gbuf=[pltpu.VMEM((CHUNK, D), jnp.float32) for _ in range(NBUF)],
        sbuf=[pltpu.VMEM((PPS, SEQ, D), jnp.float32) for _ in range(NBUF)],
        gsem=[pltpu.SemaphoreType.DMA for _ in range(NBUF)],
        ssem=[pltpu.SemaphoreType.DMA for _ in range(NBUF)],
    )

    @functools.partial(
        pl.kernel,
        mesh=mesh,
        compiler_params=pltpu.CompilerParams(use_tc_tiling_on_sc=False),
        out_type=jax.ShapeDtypeStruct((BATCH, SEQ, D), jnp.float32),
        scratch_types=scratch,
    )
    def k(x_hbm, table_hbm, out_hbm, idx_all, gbuf, sbuf, gsem, ssem):
        wid = lax.axis_index("s") * NC + lax.axis_index("c")
        rbase = wid * b_per_w
        pbase = wid * p_per_w

        pltpu.sync_copy(x_hbm.at[pl.ds(rbase, b_per_w)], idx_all)

        def issue_gather(c, b):
            pltpu.async_copy(
                table_hbm.at[idx_all.at[pl.ds(c * CHUNK, CHUNK)]],
                gbuf[b], gsem[b])

        def issue_store(c, b):
            pltpu.async_copy(
                sbuf[b], out_hbm.at[pl.ds(pbase + c * PPS, PPS)], ssem[b])

        def wait_gather(b):
            pltpu.make_async_copy(table_hbm.at[idx_all.at[pl.ds(0, CHUNK)]],
                                  gbuf[b], gsem[b]).wait()

        def wait_store(b):
            pltpu.make_async_copy(sbuf[b], out_hbm.at[pl.ds(0, PPS)],
                                  ssem[b]).wait()

        def scale(b):
            for p in range(PPS):
                @plsc.parallel_loop(0, SEQ, unroll=8)
                def row_body(s):
                    for j in range(D // L):
                        sl = pl.ds(j * L, L)
                        sbuf[b][p, s, sl] = gbuf[b][p * SEQ + s, sl] * SCALE

        for b in range(NBUF):
            issue_gather(b, b)

        for b in range(NBUF):
            wait_gather(b)
            scale(b)
            issue_gather(NBUF + b, b)
            issue_store(b, b)

        def round_body(r, carry):
            c0 = r * NBUF
            for b in range(NBUF):
                c = c0 + b
                wait_gather(b)
                wait_store(b)
                scale(b)
                issue_gather(c + NBUF, b)
                issue_store(c, b)
            return carry
        lax.fori_loop(1, n_rounds - 1, round_body, 0)

        for b in range(NBUF):
            c = (n_rounds - 1) * NBUF + b
            wait_gather(b)
            wait_store(b)
            scale(b)
            issue_store(c, b)
        for b in range(NBUF):
            wait_store(b)

    return k


def kernel(x, table):
    BATCH, SEQ = x.shape
    xf = x.reshape(BATCH * SEQ).astype(jnp.int32)
    return _make_kernel(BATCH, SEQ)(xf, table)
